# baseline (speedup 1.0000x reference)
import functools

import jax
import jax.numpy as jnp
from jax import lax
from jax.experimental import pallas as pl
from jax.experimental.pallas import tpu as pltpu
from jax.experimental.pallas import tpu_sc as plsc

B = 8
P = 4096
N1, N2, N3 = 1024, 256, 64
MIN_ALPHA = 0.01
_HI = jax.lax.Precision.HIGHEST
_INTERPRET = False
_USE_SC = False


def _dot(a, b, prec=jax.lax.Precision.DEFAULT):
    return jax.lax.dot_general(a, b, (((1,), (0,)), ((), ())),
                               precision=prec, preferred_element_type=jnp.float32)


def _iota(shape, dim):
    return jax.lax.broadcasted_iota(jnp.int32, shape, dim)


def _merge(a, b):
    va, ia, xa, ya, za = a
    vb, ib, xb, yb, zb = b
    ta = jnp.logical_or(va > vb, jnp.logical_and(va == vb, ia < ib))
    return (jnp.where(ta, va, vb), jnp.where(ta, ia, ib),
            jnp.where(ta, xa, xb), jnp.where(ta, ya, yb),
            jnp.where(ta, za, zb))


def _fps_level(x, y, z, n, idx_ref, sx_ref, sy_ref, sz_ref):
    m = x.shape[1]
    nch = m // 128
    lane128 = _iota((1, 128), 1)
    sls = [slice(j * 128, (j + 1) * 128) for j in range(nch)]
    xs = [x[:, s] for s in sls]
    ys = [y[:, s] for s in sls]
    zs = [z[:, s] for s in sls]
    x0, y0, z0 = x[:, 0:1], y[:, 0:1], z[:, 0:1]

    def dist_chunks(cx, cy, cz):
        out = []
        for j in range(nch):
            dx, dy, dz = xs[j] - cx, ys[j] - cy, zs[j] - cz
            out.append(dx * dx + dy * dy + dz * dz)
        return out

    d0 = dist_chunks(x0, y0, z0)

    ch = min(n, 128)
    lane_ch = _iota((1, ch), 1)

    def step_fn(c):
        def step(t, carry):
            ds, axx, ayy, azz, aidx = carry
            ds = list(ds)
            cand = [(ds[j], lane128 + j * 128, xs[j], ys[j], zs[j])
                    for j in range(nch)]
            while len(cand) > 1:
                nxt_l = [_merge(cand[i], cand[i + 1])
                         for i in range(0, len(cand) - 1, 2)]
                if len(cand) % 2:
                    nxt_l.append(cand[-1])
                cand = nxt_l
            best = cand[0]
            shift = 64
            while shift >= 1:
                rolled = tuple(pltpu.roll(t_, shift, 1) for t_ in best)
                best = _merge(best, rolled)
                shift //= 2
            _, bi, bx, by, bz = best
            for j in range(nch):
                dx, dy, dz = xs[j] - bx, ys[j] - by, zs[j] - bz
                ds[j] = jnp.minimum(ds[j], dx * dx + dy * dy + dz * dz)
            sel = lane_ch == (t - c * ch)
            axx = jnp.where(sel, bx[:, 0:1], axx)
            ayy = jnp.where(sel, by[:, 0:1], ayy)
            azz = jnp.where(sel, bz[:, 0:1], azz)
            aidx = jnp.where(sel, bi[:, 0:1], aidx)
            return tuple(ds), axx, ayy, azz, aidx

        return step

    ds = tuple(d0)
    for c in range(n // ch):
        if c == 0:
            axx = jnp.where(lane_ch == 0, x0, 0.0)
            ayy = jnp.where(lane_ch == 0, y0, 0.0)
            azz = jnp.where(lane_ch == 0, z0, 0.0)
            lo = 1
        else:
            axx = jnp.zeros((B, ch), jnp.float32)
            ayy = jnp.zeros((B, ch), jnp.float32)
            azz = jnp.zeros((B, ch), jnp.float32)
            lo = c * ch
        aidx = jnp.zeros((B, ch), jnp.int32)
        ds, axx, ayy, azz, aidx = jax.lax.fori_loop(
            lo, (c + 1) * ch, step_fn(c), (ds, axx, ayy, azz, aidx))
        sl = slice(c * ch, (c + 1) * ch)
        sx_ref[:, sl] = axx
        sy_ref[:, sl] = ayy
        sz_ref[:, sl] = azz
        if idx_ref is not None:
            idx_ref[:, sl] = aidx


def _fps_body(px, py, pz,
              idx2_ref, idx3_ref,
              l1x_ref, l1y_ref, l1z_ref,
              l2x_ref, l2y_ref, l2z_ref,
              l3x_ref, l3y_ref, l3z_ref):
    x, y, z = px[...], py[...], pz[...]
    _fps_level(x, y, z, N1, None, l1x_ref, l1y_ref, l1z_ref)
    x1, y1, z1 = l1x_ref[...], l1y_ref[...], l1z_ref[...]
    _fps_level(x1, y1, z1, N2, idx2_ref, l2x_ref, l2y_ref, l2z_ref)
    x2, y2, z2 = l2x_ref[...], l2y_ref[...], l2z_ref[...]
    _fps_level(x2, y2, z2, N3, idx3_ref, l3x_ref, l3y_ref, l3z_ref)


def _fps_call(px, py, pz):
    f32, i32 = jnp.float32, jnp.int32
    outs = [
        jax.ShapeDtypeStruct((B, N2), i32),
        jax.ShapeDtypeStruct((B, N3), i32),
        jax.ShapeDtypeStruct((B, N1), f32), jax.ShapeDtypeStruct((B, N1), f32),
        jax.ShapeDtypeStruct((B, N1), f32),
        jax.ShapeDtypeStruct((B, N2), f32), jax.ShapeDtypeStruct((B, N2), f32),
        jax.ShapeDtypeStruct((B, N2), f32),
        jax.ShapeDtypeStruct((B, N3), f32), jax.ShapeDtypeStruct((B, N3), f32),
        jax.ShapeDtypeStruct((B, N3), f32),
    ]
    return pl.pallas_call(_fps_body, out_shape=outs, interpret=_INTERPRET)(px, py, pz)


def _sa1_body(lx, ly, lz, w0, b0, w1, b1, w2, b2, out):
    h = lx[...] * w0[0:1, :] + ly[...] * w0[1:2, :] + lz[...] * w0[2:3, :] + b0[...]
    h = jnp.maximum(h, 0.0)
    h = jnp.maximum(_dot(h, w1[...]) + b1[...], 0.0)
    out[...] = jnp.maximum(_dot(h, w2[...]) + b2[...], 0.0)


def _sa1_call(lx, ly, lz, w0, b0, w1, b1, w2, b2):
    n = lx.shape[0]
    return pl.pallas_call(
        _sa1_body,
        out_shape=jax.ShapeDtypeStruct((n, 128), jnp.float32),
        interpret=_INTERPRET,
    )(lx, ly, lz, w0, b0, w1, b1, w2, b2)


def _sa_gather_body(idx, feat, qx, qy, qz,
                    w0f, w0p, b0, w1, b1, w2, b2, out, *, nsrc):
    nq = idx.shape[1]
    oneh = (_iota((1, nsrc), 1) == idx[0]).astype(jnp.float32)
    g = _dot(oneh, feat[0], _HI)
    posw = (qx[0] * w0p[0:1, :] + qy[0] * w0p[1:2, :] + qz[0] * w0p[2:3, :])
    h = jnp.maximum(_dot(g, w0f[...]) + posw + b0[...], 0.0)
    h = jnp.maximum(_dot(h, w1[...]) + b1[...], 0.0)
    out[0] = jnp.maximum(_dot(h, w2[...]) + b2[...], 0.0)


def _sa_gather_call(idx, feat, qx, qy, qz, w0f, w0p, b0, w1, b1, w2, b2):
    nq = idx.shape[1]
    nsrc, cin = feat.shape[1], feat.shape[2]
    cout = w2.shape[1]
    cmid1, cmid2 = w0f.shape[1], w1.shape[1]
    spec_w = lambda s: pl.BlockSpec(s, lambda b: (0,) * len(s))
    return pl.pallas_call(
        functools.partial(_sa_gather_body, nsrc=nsrc),
        grid=(B,),
        in_specs=[
            pl.BlockSpec((1, nq, 1), lambda b: (b, 0, 0)),
            pl.BlockSpec((1, nsrc, cin), lambda b: (b, 0, 0)),
            pl.BlockSpec((1, nq, 1), lambda b: (b, 0, 0)),
            pl.BlockSpec((1, nq, 1), lambda b: (b, 0, 0)),
            pl.BlockSpec((1, nq, 1), lambda b: (b, 0, 0)),
            spec_w((cin, cmid1)), spec_w((3, cmid1)), spec_w((1, cmid1)),
            spec_w((cmid1, cmid2)), spec_w((1, cmid2)),
            spec_w((cmid2, cout)), spec_w((1, cout)),
        ],
        out_specs=pl.BlockSpec((1, nq, cout), lambda b: (b, 0, 0)),
        out_shape=jax.ShapeDtypeStruct((B, nq, cout), jnp.float32),
        interpret=_INTERPRET,
    )(idx, feat, qx, qy, qz, w0f, w0p, b0, w1, b1, w2, b2)


def _knn_weights(qx, qy, qz, sx, sy, sz):
    dx, dy, dz = qx - sx, qy - sy, qz - sz
    d = dx * dx + dy * dy + dz * dz
    nq, ns = d.shape
    lane = _iota((1, ns), 1)
    mat = jnp.zeros((nq, ns), jnp.float32)
    wt = jnp.zeros((nq, 1), jnp.float32)
    for k in range(3):
        m = jnp.min(d, axis=1, keepdims=True)
        j = jnp.min(jnp.where(d == m, lane, ns), axis=1, keepdims=True)
        oneh = lane == j
        w = 1.0 / jnp.maximum(m, 1e-16)
        mat = jnp.where(oneh, w, mat)
        wt = wt + w
        if k < 2:
            d = jnp.where(oneh, jnp.inf, d)
    return mat, wt


def _fp_body(qx, qy, qz, sx, sy, sz, srcf, skipf,
             w0a, w0b, b0, w1, b1, out):
    mat, wt = _knn_weights(qx[0], qy[0], qz[0], sx[0], sy[0], sz[0])
    interp = _dot(mat, srcf[0]) / wt
    h = jnp.maximum(_dot(interp, w0a[...]) + _dot(skipf[0], w0b[...]) + b0[...], 0.0)
    out[0] = jnp.maximum(_dot(h, w1[...]) + b1[...], 0.0)


def _fp_call(qpos, spos, srcf, skipf, w0a, w0b, b0, w1, b1):
    qx, qy, qz = qpos
    sx, sy, sz = spos
    nq, ns = qx.shape[1], sx.shape[2]
    cin, cskip = srcf.shape[2], skipf.shape[2]
    cmid, cout = w0a.shape[1], w1.shape[1]
    spec_w = lambda s: pl.BlockSpec(s, lambda b: (0,) * len(s))
    return pl.pallas_call(
        _fp_body,
        grid=(B,),
        in_specs=[
            pl.BlockSpec((1, nq, 1), lambda b: (b, 0, 0)),
            pl.BlockSpec((1, nq, 1), lambda b: (b, 0, 0)),
            pl.BlockSpec((1, nq, 1), lambda b: (b, 0, 0)),
            pl.BlockSpec((1, 1, ns), lambda b: (b, 0, 0)),
            pl.BlockSpec((1, 1, ns), lambda b: (b, 0, 0)),
            pl.BlockSpec((1, 1, ns), lambda b: (b, 0, 0)),
            pl.BlockSpec((1, ns, cin), lambda b: (b, 0, 0)),
            pl.BlockSpec((1, nq, cskip), lambda b: (b, 0, 0)),
            spec_w((cin, cmid)), spec_w((cskip, cmid)), spec_w((1, cmid)),
            spec_w((cmid, cout)), spec_w((1, cout)),
        ],
        out_specs=pl.BlockSpec((1, nq, cout), lambda b: (b, 0, 0)),
        out_shape=jax.ShapeDtypeStruct((B, nq, cout), jnp.float32),
        interpret=_INTERPRET,
    )(qx, qy, qz, sx, sy, sz, srcf, skipf, w0a, w0b, b0, w1, b1)


def _knn1_body(qx, qy, qz, sx, sy, sz, idx_ref, w_ref):
    b = pl.program_id(0)
    dx = qx[0] - sx[0]
    dy = qy[0] - sy[0]
    dz = qz[0] - sz[0]
    d = dx * dx + dy * dy + dz * dz
    nq, ns = d.shape
    lane = _iota((1, ns), 1)
    wt = jnp.zeros((nq, 1), jnp.float32)
    js, ws = [], []
    for k in range(3):
        m = jnp.min(d, axis=1, keepdims=True)
        j = jnp.min(jnp.where(d == m, lane, ns), axis=1, keepdims=True)
        w = 1.0 / jnp.maximum(m, 1e-16)
        js.append(j)
        ws.append(w)
        wt = wt + w
        if k < 2:
            d = jnp.where(lane == j, jnp.inf, d)
    for k in range(3):
        idx_ref[0, :, k:k + 1] = js[k] + b * ns
        w_ref[0, :, k:k + 1] = ws[k] / wt


def _knn1_call(qpos, spos):
    qx, qy, qz = qpos
    sx, sy, sz = spos
    qt = 1024
    nblk = P // qt
    outs = [
        jax.ShapeDtypeStruct((B, P, 3), jnp.int32),
        jax.ShapeDtypeStruct((B, P, 3), jnp.float32),
    ]
    return pl.pallas_call(
        _knn1_body,
        grid=(B, nblk),
        in_specs=[
            pl.BlockSpec((1, qt, 1), lambda b, j: (b, j, 0)),
            pl.BlockSpec((1, qt, 1), lambda b, j: (b, j, 0)),
            pl.BlockSpec((1, qt, 1), lambda b, j: (b, j, 0)),
            pl.BlockSpec((1, 1, N1), lambda b, j: (b, 0, 0)),
            pl.BlockSpec((1, 1, N1), lambda b, j: (b, 0, 0)),
            pl.BlockSpec((1, 1, N1), lambda b, j: (b, 0, 0)),
        ],
        out_specs=[
            pl.BlockSpec((1, qt, 3), lambda b, j: (b, j, 0)),
            pl.BlockSpec((1, qt, 3), lambda b, j: (b, j, 0)),
        ],
        out_shape=outs,
        interpret=_INTERPRET,
    )(qx, qy, qz, sx, sy, sz)


_NW = 32
_QC = 128


def _interp_sc_call(table, idxg, wts):
    nq = idxg.shape[0] // 3
    qpw = nq // _NW
    nch = qpw // _QC
    mesh = plsc.VectorSubcoreMesh(core_axis_name="c", subcore_axis_name="s")

    @functools.partial(
        pl.kernel, mesh=mesh,
        out_type=jax.ShapeDtypeStruct((nq, 128), jnp.float32),
        scratch_types=[
            pltpu.VMEM((_QC * 3,), jnp.int32),
            pltpu.VMEM((_QC * 3 + 16,), jnp.float32),
            pltpu.VMEM((_QC * 3, 128), jnp.float32),
            pltpu.VMEM((_QC, 128), jnp.float32),
            pltpu.SemaphoreType.DMA,
        ],
    )
    def k(table_h, idx_h, w_h, out_h, idx_v, w_v, rows_v, out_v, sem):
        wid = lax.axis_index("s") * 2 + lax.axis_index("c")

        def chunk(c, carry):
            qbase = wid * qpw + c * _QC
            ibase = qbase * 3
            pltpu.sync_copy(idx_h.at[pl.ds(ibase, _QC * 3)], idx_v)
            pltpu.sync_copy(w_h.at[pl.ds(ibase, _QC * 3)], w_v.at[pl.ds(0, _QC * 3)])
            pltpu.async_copy(table_h.at[idx_v], rows_v, sem).wait()

            def q_loop(q, carry2):
                wv = w_v[pl.ds(3 * q, 16)]
                w0 = wv[0]
                w1 = wv[1]
                w2 = wv[2]
                for s in range(8):
                    sl = pl.ds(s * 16, 16)
                    acc = (rows_v[3 * q, sl] * w0
                           + rows_v[3 * q + 1, sl] * w1
                           + rows_v[3 * q + 2, sl] * w2)
                    out_v[q, sl] = acc
                return carry2

            lax.fori_loop(0, _QC, q_loop, 0)
            pltpu.sync_copy(out_v, out_h.at[pl.ds(qbase, _QC)])
            return carry

        lax.fori_loop(0, nch, chunk, 0)

    return k(table, idxg, wts)


def _fp1mlp_body(interp, qx, qy, qz,
                 w0f, w0p, b0, w1, b1, w2, b2, wh0, bh0, wh1, bh1, out):
    posw = (qx[0] * w0p[0:1, :] + qy[0] * w0p[1:2, :] + qz[0] * w0p[2:3, :])
    h = jnp.maximum(_dot(interp[0], w0f[...]) + posw + b0[...], 0.0)
    h = jnp.maximum(_dot(h, w1[...]) + b1[...], 0.0)
    h = jnp.maximum(_dot(h, w2[...]) + b2[...], 0.0)
    a = jnp.maximum(_dot(h, wh0[...]) + bh0[...], 0.0)
    v = _dot(a, wh1[...]) + bh1[...]
    sp = jnp.maximum(v, 0.0) + jnp.log(1.0 + jnp.exp(-jnp.abs(v)))
    out[0] = sp + MIN_ALPHA


def _fp1mlp_call(interp, qpos, wf, hd):
    qx, qy, qz = qpos
    qt = 1024
    nblk = P // qt
    spec_w = lambda s: pl.BlockSpec(s, lambda b, j: (0,) * len(s))
    w0f, w0p, b0, w1, b1, w2, b2 = wf
    wh0, bh0, wh1, bh1 = hd
    return pl.pallas_call(
        _fp1mlp_body,
        grid=(B, nblk),
        in_specs=[
            pl.BlockSpec((1, qt, 128), lambda b, j: (b, j, 0)),
            pl.BlockSpec((1, qt, 1), lambda b, j: (b, j, 0)),
            pl.BlockSpec((1, qt, 1), lambda b, j: (b, j, 0)),
            pl.BlockSpec((1, qt, 1), lambda b, j: (b, j, 0)),
            spec_w((128, 128)), spec_w((3, 128)), spec_w((1, 128)),
            spec_w((128, 128)), spec_w((1, 128)),
            spec_w((128, 128)), spec_w((1, 128)),
            spec_w((128, 64)), spec_w((1, 64)),
            spec_w((64, 1)), spec_w((1, 1)),
        ],
        out_specs=pl.BlockSpec((1, qt, 1), lambda b, j: (b, j, 0)),
        out_shape=jax.ShapeDtypeStruct((B, P, 1), jnp.float32),
        interpret=_INTERPRET,
    )(interp, qx, qy, qz,
      w0f, w0p, b0, w1, b1, w2, b2, wh0, bh0, wh1, bh1)


def _fp1_body(qx, qy, qz, sx, sy, sz, srcf,
              w0f, w0p, b0, w1, b1, w2, b2, wh0, bh0, wh1, bh1, out):
    mat, wt = _knn_weights(qx[0], qy[0], qz[0], sx[0], sy[0], sz[0])
    interp = _dot(mat, srcf[0]) / wt
    posw = (qx[0] * w0p[0:1, :] + qy[0] * w0p[1:2, :] + qz[0] * w0p[2:3, :])
    h = jnp.maximum(_dot(interp, w0f[...]) + posw + b0[...], 0.0)
    h = jnp.maximum(_dot(h, w1[...]) + b1[...], 0.0)
    h = jnp.maximum(_dot(h, w2[...]) + b2[...], 0.0)
    a = jnp.maximum(_dot(h, wh0[...]) + bh0[...], 0.0)
    v = _dot(a, wh1[...]) + bh1[...]
    sp = jnp.maximum(v, 0.0) + jnp.log(1.0 + jnp.exp(-jnp.abs(v)))
    out[0] = sp + MIN_ALPHA


def _fp1_call(qpos, spos, srcf, wf, hd):
    qx, qy, qz = qpos
    sx, sy, sz = spos
    qt = 1024
    nblk = P // qt
    spec_w = lambda s: pl.BlockSpec(s, lambda b, j: (0,) * len(s))
    w0f, w0p, b0, w1, b1, w2, b2 = wf
    wh0, bh0, wh1, bh1 = hd
    return pl.pallas_call(
        _fp1_body,
        grid=(B, nblk),
        in_specs=[
            pl.BlockSpec((1, qt, 1), lambda b, j: (b, j, 0)),
            pl.BlockSpec((1, qt, 1), lambda b, j: (b, j, 0)),
            pl.BlockSpec((1, qt, 1), lambda b, j: (b, j, 0)),
            pl.BlockSpec((1, 1, N1), lambda b, j: (b, 0, 0)),
            pl.BlockSpec((1, 1, N1), lambda b, j: (b, 0, 0)),
            pl.BlockSpec((1, 1, N1), lambda b, j: (b, 0, 0)),
            pl.BlockSpec((1, N1, 128), lambda b, j: (b, 0, 0)),
            spec_w((128, 128)), spec_w((3, 128)), spec_w((1, 128)),
            spec_w((128, 128)), spec_w((1, 128)),
            spec_w((128, 128)), spec_w((1, 128)),
            spec_w((128, 64)), spec_w((1, 64)),
            spec_w((64, 1)), spec_w((1, 1)),
        ],
        out_specs=pl.BlockSpec((1, qt, 1), lambda b, j: (b, j, 0)),
        out_shape=jax.ShapeDtypeStruct((B, P, 1), jnp.float32),
        interpret=_INTERPRET,
    )(qx, qy, qz, sx, sy, sz, srcf,
      w0f, w0p, b0, w1, b1, w2, b2, wh0, bh0, wh1, bh1)


def kernel(pos, batch, params):
    del batch
    f32 = jnp.float32
    pos_b = pos.reshape(B, P, 3)
    px = pos_b[..., 0]
    py = pos_b[..., 1]
    pz = pos_b[..., 2]

    (idx2, idx3, l1x, l1y, l1z, l2x, l2y, l2z, l3x, l3y, l3z) = _fps_call(px, py, pz)

    rb = lambda b_: b_.reshape(1, -1).astype(f32)
    sa1 = params["sa1"]
    l1feat = _sa1_call(
        l1x.reshape(-1, 1), l1y.reshape(-1, 1), l1z.reshape(-1, 1),
        sa1["W"][0], rb(sa1["b"][0]), sa1["W"][1], rb(sa1["b"][1]),
        sa1["W"][2], rb(sa1["b"][2]),
    ).reshape(B, N1, 128)

    sa2 = params["sa2"]
    l2feat = _sa_gather_call(
        idx2.reshape(B, N2, 1), l1feat,
        l2x.reshape(B, N2, 1), l2y.reshape(B, N2, 1), l2z.reshape(B, N2, 1),
        sa2["W"][0][:128], sa2["W"][0][128:], rb(sa2["b"][0]),
        sa2["W"][1], rb(sa2["b"][1]), sa2["W"][2], rb(sa2["b"][2]),
    )

    sa3 = params["sa3"]
    l3feat = _sa_gather_call(
        idx3.reshape(B, N3, 1), l2feat,
        l3x.reshape(B, N3, 1), l3y.reshape(B, N3, 1), l3z.reshape(B, N3, 1),
        sa3["W"][0][:256], sa3["W"][0][256:], rb(sa3["b"][0]),
        sa3["W"][1], rb(sa3["b"][1]), sa3["W"][2], rb(sa3["b"][2]),
    )

    fp3 = params["fp3"]
    l2fp = _fp_call(
        (l2x.reshape(B, N2, 1), l2y.reshape(B, N2, 1), l2z.reshape(B, N2, 1)),
        (l3x.reshape(B, 1, N3), l3y.reshape(B, 1, N3), l3z.reshape(B, 1, N3)),
        l3feat, l2feat,
        fp3["W"][0][:1024], fp3["W"][0][1024:], rb(fp3["b"][0]),
        fp3["W"][1], rb(fp3["b"][1]),
    )

    fp2 = params["fp2"]
    l1fp = _fp_call(
        (l1x.reshape(B, N1, 1), l1y.reshape(B, N1, 1), l1z.reshape(B, N1, 1)),
        (l2x.reshape(B, 1, N2), l2y.reshape(B, 1, N2), l2z.reshape(B, 1, N2)),
        l2fp, l1feat,
        fp2["W"][0][:256], fp2["W"][0][256:], rb(fp2["b"][0]),
        fp2["W"][1], rb(fp2["b"][1]),
    )

    fp1 = params["fp1"]
    hd = params["head"]
    fp1_w = (fp1["W"][0][:128], fp1["W"][0][128:], rb(fp1["b"][0]),
             fp1["W"][1], rb(fp1["b"][1]), fp1["W"][2], rb(fp1["b"][2]))
    hd_w = (hd["W"][0], rb(hd["b"][0]), hd["W"][1], rb(hd["b"][1]))
    qpos0 = (pos_b[..., 0:1], pos_b[..., 1:2], pos_b[..., 2:3])
    spos1 = (l1x.reshape(B, 1, N1), l1y.reshape(B, 1, N1), l1z.reshape(B, 1, N1))
    if _USE_SC:
        idxg, wts = _knn1_call(qpos0, spos1)
        interp = _interp_sc_call(
            l1fp.reshape(B * N1, 128), idxg.reshape(-1), wts.reshape(-1))
        alpha = _fp1mlp_call(interp.reshape(B, P, 128), qpos0, fp1_w, hd_w)
    else:
        alpha = _fp1_call(qpos0, spos1, l1fp, fp1_w, hd_w)

    alpha_mean_act = alpha.reshape(B, 1, P)
    alpha_std = jnp.full_like(alpha_mean_act, 0.01)
    return alpha_mean_act, alpha_std


# speedup vs baseline: 1.3025x; 1.3025x over previous
import functools

import jax
import jax.numpy as jnp
from jax.experimental import pallas as pl

B = 8
P = 4096
N1, N2, N3 = 1024, 256, 64
MIN_ALPHA = 0.01
_HI = jax.lax.Precision.HIGHEST


def _dot(a, b, prec=jax.lax.Precision.DEFAULT):
    return jax.lax.dot_general(a, b, (((1,), (0,)), ((), ())),
                               precision=prec, preferred_element_type=jnp.float32)


def _iota(shape, dim):
    return jax.lax.broadcasted_iota(jnp.int32, shape, dim)


def _fps_level(x, y, z, n, idx_ref, sx_ref, sy_ref, sz_ref):
    m = x.shape[1]
    lane_m = _iota((1, m), 1)
    x0, y0, z0 = x[:, 0:1], y[:, 0:1], z[:, 0:1]
    dx, dy, dz = x - x0, y - y0, z - z0
    d = dx * dx + dy * dy + dz * dz

    ch = min(n, 128)
    lane_ch = _iota((1, ch), 1)

    def step_fn(c):
        def step(t, carry):
            d, axx, ayy, azz, aidx = carry
            mx = jnp.max(d, axis=1, keepdims=True)
            nxt = jnp.min(jnp.where(d == mx, lane_m, m), axis=1, keepdims=True)
            oneh = lane_m == nxt
            sx = jnp.sum(jnp.where(oneh, x, 0.0), axis=1, keepdims=True)
            sy = jnp.sum(jnp.where(oneh, y, 0.0), axis=1, keepdims=True)
            sz = jnp.sum(jnp.where(oneh, z, 0.0), axis=1, keepdims=True)
            dx, dy, dz = x - sx, y - sy, z - sz
            d = jnp.minimum(d, dx * dx + dy * dy + dz * dz)
            sel = lane_ch == (t - c * ch)
            axx = jnp.where(sel, sx, axx)
            ayy = jnp.where(sel, sy, ayy)
            azz = jnp.where(sel, sz, azz)
            aidx = jnp.where(sel, nxt, aidx)
            return d, axx, ayy, azz, aidx

        return step

    for c in range(n // ch):
        if c == 0:
            axx = jnp.where(lane_ch == 0, x0, 0.0)
            ayy = jnp.where(lane_ch == 0, y0, 0.0)
            azz = jnp.where(lane_ch == 0, z0, 0.0)
            lo = 1
        else:
            axx = jnp.zeros((B, ch), jnp.float32)
            ayy = jnp.zeros((B, ch), jnp.float32)
            azz = jnp.zeros((B, ch), jnp.float32)
            lo = c * ch
        aidx = jnp.zeros((B, ch), jnp.int32)
        d, axx, ayy, azz, aidx = jax.lax.fori_loop(
            lo, (c + 1) * ch, step_fn(c), (d, axx, ayy, azz, aidx))
        sl = slice(c * ch, (c + 1) * ch)
        sx_ref[:, sl] = axx
        sy_ref[:, sl] = ayy
        sz_ref[:, sl] = azz
        if idx_ref is not None:
            idx_ref[:, sl] = aidx


def _fps_body(px, py, pz,
              idx2_ref, idx3_ref,
              l1x_ref, l1y_ref, l1z_ref,
              l2x_ref, l2y_ref, l2z_ref,
              l3x_ref, l3y_ref, l3z_ref):
    x, y, z = px[...], py[...], pz[...]
    _fps_level(x, y, z, N1, None, l1x_ref, l1y_ref, l1z_ref)
    x1, y1, z1 = l1x_ref[...], l1y_ref[...], l1z_ref[...]
    _fps_level(x1, y1, z1, N2, idx2_ref, l2x_ref, l2y_ref, l2z_ref)
    x2, y2, z2 = l2x_ref[...], l2y_ref[...], l2z_ref[...]
    _fps_level(x2, y2, z2, N3, idx3_ref, l3x_ref, l3y_ref, l3z_ref)


def _fps_call(px, py, pz):
    f32, i32 = jnp.float32, jnp.int32
    outs = [
        jax.ShapeDtypeStruct((B, N2), i32),
        jax.ShapeDtypeStruct((B, N3), i32),
        jax.ShapeDtypeStruct((B, N1), f32), jax.ShapeDtypeStruct((B, N1), f32),
        jax.ShapeDtypeStruct((B, N1), f32),
        jax.ShapeDtypeStruct((B, N2), f32), jax.ShapeDtypeStruct((B, N2), f32),
        jax.ShapeDtypeStruct((B, N2), f32),
        jax.ShapeDtypeStruct((B, N3), f32), jax.ShapeDtypeStruct((B, N3), f32),
        jax.ShapeDtypeStruct((B, N3), f32),
    ]
    return pl.pallas_call(_fps_body, out_shape=outs)(px, py, pz)


def _sa1_body(lx, ly, lz, w0, b0, w1, b1, w2, b2, out):
    h = lx[...] * w0[0:1, :] + ly[...] * w0[1:2, :] + lz[...] * w0[2:3, :] + b0[...]
    h = jnp.maximum(h, 0.0)
    h = jnp.maximum(_dot(h, w1[...]) + b1[...], 0.0)
    out[...] = jnp.maximum(_dot(h, w2[...]) + b2[...], 0.0)


def _sa1_call(lx, ly, lz, w0, b0, w1, b1, w2, b2):
    n = lx.shape[0]
    return pl.pallas_call(
        _sa1_body,
        out_shape=jax.ShapeDtypeStruct((n, 128), jnp.float32),
    )(lx, ly, lz, w0, b0, w1, b1, w2, b2)


def _sa_gather_body(idx, feat, qx, qy, qz,
                    w0f, w0p, b0, w1, b1, w2, b2, out, *, nsrc):
    nq = idx.shape[1]
    oneh = (_iota((1, nsrc), 1) == idx[0]).astype(jnp.float32)
    g = _dot(oneh, feat[0], _HI)
    posw = (qx[0] * w0p[0:1, :] + qy[0] * w0p[1:2, :] + qz[0] * w0p[2:3, :])
    h = jnp.maximum(_dot(g, w0f[...]) + posw + b0[...], 0.0)
    h = jnp.maximum(_dot(h, w1[...]) + b1[...], 0.0)
    out[0] = jnp.maximum(_dot(h, w2[...]) + b2[...], 0.0)


def _sa_gather_call(idx, feat, qx, qy, qz, w0f, w0p, b0, w1, b1, w2, b2):
    nq = idx.shape[1]
    nsrc, cin = feat.shape[1], feat.shape[2]
    cout = w2.shape[1]
    cmid1, cmid2 = w0f.shape[1], w1.shape[1]
    spec_w = lambda s: pl.BlockSpec(s, lambda b: (0,) * len(s))
    return pl.pallas_call(
        functools.partial(_sa_gather_body, nsrc=nsrc),
        grid=(B,),
        in_specs=[
            pl.BlockSpec((1, nq, 1), lambda b: (b, 0, 0)),
            pl.BlockSpec((1, nsrc, cin), lambda b: (b, 0, 0)),
            pl.BlockSpec((1, nq, 1), lambda b: (b, 0, 0)),
            pl.BlockSpec((1, nq, 1), lambda b: (b, 0, 0)),
            pl.BlockSpec((1, nq, 1), lambda b: (b, 0, 0)),
            spec_w((cin, cmid1)), spec_w((3, cmid1)), spec_w((1, cmid1)),
            spec_w((cmid1, cmid2)), spec_w((1, cmid2)),
            spec_w((cmid2, cout)), spec_w((1, cout)),
        ],
        out_specs=pl.BlockSpec((1, nq, cout), lambda b: (b, 0, 0)),
        out_shape=jax.ShapeDtypeStruct((B, nq, cout), jnp.float32),
    )(idx, feat, qx, qy, qz, w0f, w0p, b0, w1, b1, w2, b2)


def _knn_weights(qx, qy, qz, sx, sy, sz):
    dx, dy, dz = qx - sx, qy - sy, qz - sz
    d = dx * dx + dy * dy + dz * dz
    nq, ns = d.shape
    lane = _iota((1, ns), 1)
    mat = jnp.zeros((nq, ns), jnp.float32)
    wt = jnp.zeros((nq, 1), jnp.float32)
    for k in range(3):
        m = jnp.min(d, axis=1, keepdims=True)
        j = jnp.min(jnp.where(d == m, lane, ns), axis=1, keepdims=True)
        oneh = lane == j
        w = 1.0 / jnp.maximum(m, 1e-16)
        mat = jnp.where(oneh, w, mat)
        wt = wt + w
        if k < 2:
            d = jnp.where(oneh, jnp.inf, d)
    return mat, wt


def _fp_body(qx, qy, qz, sx, sy, sz, srcf, skipf,
             w0a, w0b, b0, w1, b1, out):
    mat, wt = _knn_weights(qx[0], qy[0], qz[0], sx[0], sy[0], sz[0])
    interp = _dot(mat, srcf[0]) / wt
    h = jnp.maximum(_dot(interp, w0a[...]) + _dot(skipf[0], w0b[...]) + b0[...], 0.0)
    out[0] = jnp.maximum(_dot(h, w1[...]) + b1[...], 0.0)


def _fp_call(qpos, spos, srcf, skipf, w0a, w0b, b0, w1, b1):
    qx, qy, qz = qpos
    sx, sy, sz = spos
    nq, ns = qx.shape[1], sx.shape[2]
    cin, cskip = srcf.shape[2], skipf.shape[2]
    cmid, cout = w0a.shape[1], w1.shape[1]
    spec_w = lambda s: pl.BlockSpec(s, lambda b: (0,) * len(s))
    return pl.pallas_call(
        _fp_body,
        grid=(B,),
        in_specs=[
            pl.BlockSpec((1, nq, 1), lambda b: (b, 0, 0)),
            pl.BlockSpec((1, nq, 1), lambda b: (b, 0, 0)),
            pl.BlockSpec((1, nq, 1), lambda b: (b, 0, 0)),
            pl.BlockSpec((1, 1, ns), lambda b: (b, 0, 0)),
            pl.BlockSpec((1, 1, ns), lambda b: (b, 0, 0)),
            pl.BlockSpec((1, 1, ns), lambda b: (b, 0, 0)),
            pl.BlockSpec((1, ns, cin), lambda b: (b, 0, 0)),
            pl.BlockSpec((1, nq, cskip), lambda b: (b, 0, 0)),
            spec_w((cin, cmid)), spec_w((cskip, cmid)), spec_w((1, cmid)),
            spec_w((cmid, cout)), spec_w((1, cout)),
        ],
        out_specs=pl.BlockSpec((1, nq, cout), lambda b: (b, 0, 0)),
        out_shape=jax.ShapeDtypeStruct((B, nq, cout), jnp.float32),
    )(qx, qy, qz, sx, sy, sz, srcf, skipf, w0a, w0b, b0, w1, b1)


def _fp1_body(qx, qy, qz, sx, sy, sz, srcf,
              w0f, w0p, b0, w1, b1, w2, b2, wh0, bh0, wh1, bh1, out):
    mat, wt = _knn_weights(qx[0], qy[0], qz[0], sx[0], sy[0], sz[0])
    interp = _dot(mat, srcf[0]) / wt
    posw = (qx[0] * w0p[0:1, :] + qy[0] * w0p[1:2, :] + qz[0] * w0p[2:3, :])
    h = jnp.maximum(_dot(interp, w0f[...]) + posw + b0[...], 0.0)
    h = jnp.maximum(_dot(h, w1[...]) + b1[...], 0.0)
    h = jnp.maximum(_dot(h, w2[...]) + b2[...], 0.0)
    a = jnp.maximum(_dot(h, wh0[...]) + bh0[...], 0.0)
    v = _dot(a, wh1[...]) + bh1[...]
    sp = jnp.maximum(v, 0.0) + jnp.log(1.0 + jnp.exp(-jnp.abs(v)))
    out[0] = sp + MIN_ALPHA


def _fp1_call(qpos, spos, srcf, wf, hd):
    qx, qy, qz = qpos
    sx, sy, sz = spos
    qt = 1024
    nblk = P // qt
    spec_w = lambda s: pl.BlockSpec(s, lambda b, j: (0,) * len(s))
    w0f, w0p, b0, w1, b1, w2, b2 = wf
    wh0, bh0, wh1, bh1 = hd
    return pl.pallas_call(
        _fp1_body,
        grid=(B, nblk),
        in_specs=[
            pl.BlockSpec((1, qt, 1), lambda b, j: (b, j, 0)),
            pl.BlockSpec((1, qt, 1), lambda b, j: (b, j, 0)),
            pl.BlockSpec((1, qt, 1), lambda b, j: (b, j, 0)),
            pl.BlockSpec((1, 1, N1), lambda b, j: (b, 0, 0)),
            pl.BlockSpec((1, 1, N1), lambda b, j: (b, 0, 0)),
            pl.BlockSpec((1, 1, N1), lambda b, j: (b, 0, 0)),
            pl.BlockSpec((1, N1, 128), lambda b, j: (b, 0, 0)),
            spec_w((128, 128)), spec_w((3, 128)), spec_w((1, 128)),
            spec_w((128, 128)), spec_w((1, 128)),
            spec_w((128, 128)), spec_w((1, 128)),
            spec_w((128, 64)), spec_w((1, 64)),
            spec_w((64, 1)), spec_w((1, 1)),
        ],
        out_specs=pl.BlockSpec((1, qt, 1), lambda b, j: (b, j, 0)),
        out_shape=jax.ShapeDtypeStruct((B, P, 1), jnp.float32),
    )(qx, qy, qz, sx, sy, sz, srcf,
      w0f, w0p, b0, w1, b1, w2, b2, wh0, bh0, wh1, bh1)


def kernel(pos, batch, params):
    del batch
    f32 = jnp.float32
    pos_b = pos.reshape(B, P, 3)
    px = pos_b[..., 0]
    py = pos_b[..., 1]
    pz = pos_b[..., 2]

    (idx2, idx3, l1x, l1y, l1z, l2x, l2y, l2z, l3x, l3y, l3z) = _fps_call(px, py, pz)

    rb = lambda b_: b_.reshape(1, -1).astype(f32)
    sa1 = params["sa1"]
    l1feat = _sa1_call(
        l1x.reshape(-1, 1), l1y.reshape(-1, 1), l1z.reshape(-1, 1),
        sa1["W"][0], rb(sa1["b"][0]), sa1["W"][1], rb(sa1["b"][1]),
        sa1["W"][2], rb(sa1["b"][2]),
    ).reshape(B, N1, 128)

    sa2 = params["sa2"]
    l2feat = _sa_gather_call(
        idx2.reshape(B, N2, 1), l1feat,
        l2x.reshape(B, N2, 1), l2y.reshape(B, N2, 1), l2z.reshape(B, N2, 1),
        sa2["W"][0][:128], sa2["W"][0][128:], rb(sa2["b"][0]),
        sa2["W"][1], rb(sa2["b"][1]), sa2["W"][2], rb(sa2["b"][2]),
    )

    sa3 = params["sa3"]
    l3feat = _sa_gather_call(
        idx3.reshape(B, N3, 1), l2feat,
        l3x.reshape(B, N3, 1), l3y.reshape(B, N3, 1), l3z.reshape(B, N3, 1),
        sa3["W"][0][:256], sa3["W"][0][256:], rb(sa3["b"][0]),
        sa3["W"][1], rb(sa3["b"][1]), sa3["W"][2], rb(sa3["b"][2]),
    )

    fp3 = params["fp3"]
    l2fp = _fp_call(
        (l2x.reshape(B, N2, 1), l2y.reshape(B, N2, 1), l2z.reshape(B, N2, 1)),
        (l3x.reshape(B, 1, N3), l3y.reshape(B, 1, N3), l3z.reshape(B, 1, N3)),
        l3feat, l2feat,
        fp3["W"][0][:1024], fp3["W"][0][1024:], rb(fp3["b"][0]),
        fp3["W"][1], rb(fp3["b"][1]),
    )

    fp2 = params["fp2"]
    l1fp = _fp_call(
        (l1x.reshape(B, N1, 1), l1y.reshape(B, N1, 1), l1z.reshape(B, N1, 1)),
        (l2x.reshape(B, 1, N2), l2y.reshape(B, 1, N2), l2z.reshape(B, 1, N2)),
        l2fp, l1feat,
        fp2["W"][0][:256], fp2["W"][0][256:], rb(fp2["b"][0]),
        fp2["W"][1], rb(fp2["b"][1]),
    )

    fp1 = params["fp1"]
    hd = params["head"]
    fp1_w = (fp1["W"][0][:128], fp1["W"][0][128:], rb(fp1["b"][0]),
             fp1["W"][1], rb(fp1["b"][1]), fp1["W"][2], rb(fp1["b"][2]))
    hd_w = (hd["W"][0], rb(hd["b"][0]), hd["W"][1], rb(hd["b"][1]))
    qpos0 = (pos_b[..., 0:1], pos_b[..., 1:2], pos_b[..., 2:3])
    spos1 = (l1x.reshape(B, 1, N1), l1y.reshape(B, 1, N1), l1z.reshape(B, 1, N1))
    alpha = _fp1_call(qpos0, spos1, l1fp, fp1_w, hd_w)

    alpha_mean_act = alpha.reshape(B, 1, P)
    alpha_std = jnp.full_like(alpha_mean_act, 0.01)
    return alpha_mean_act, alpha_std


# speedup vs baseline: 1.3118x; 1.0071x over previous
import functools

import jax
import jax.numpy as jnp
from jax.experimental import pallas as pl

B = 8
P = 4096
N1, N2, N3 = 1024, 256, 64
MIN_ALPHA = 0.01
_HI = jax.lax.Precision.HIGHEST


def _dot(a, b, prec=jax.lax.Precision.DEFAULT):
    return jax.lax.dot_general(a, b, (((1,), (0,)), ((), ())),
                               precision=prec, preferred_element_type=jnp.float32)


def _iota(shape, dim):
    return jax.lax.broadcasted_iota(jnp.int32, shape, dim)


def _fps_level(x, y, z, n, idx_ref, sx_ref, sy_ref, sz_ref):
    m = x.shape[1]
    lane_m = _iota((1, m), 1)
    x0, y0, z0 = x[:, 0:1], y[:, 0:1], z[:, 0:1]
    dx, dy, dz = x - x0, y - y0, z - z0
    d = dx * dx + dy * dy + dz * dz

    ch = min(n, 128)
    lane_ch = _iota((1, ch), 1)

    def step_fn(c):
        def step(t, carry):
            d, axx, ayy, azz, aidx = carry
            mx = jnp.max(d, axis=1, keepdims=True)
            nxt = jnp.min(jnp.where(d == mx, lane_m, m), axis=1, keepdims=True)
            oneh = lane_m == nxt
            sx = jnp.sum(jnp.where(oneh, x, 0.0), axis=1, keepdims=True)
            sy = jnp.sum(jnp.where(oneh, y, 0.0), axis=1, keepdims=True)
            sz = jnp.sum(jnp.where(oneh, z, 0.0), axis=1, keepdims=True)
            dx, dy, dz = x - sx, y - sy, z - sz
            d = jnp.minimum(d, dx * dx + dy * dy + dz * dz)
            sel = lane_ch == (t - c * ch)
            axx = jnp.where(sel, sx, axx)
            ayy = jnp.where(sel, sy, ayy)
            azz = jnp.where(sel, sz, azz)
            aidx = jnp.where(sel, nxt, aidx)
            return d, axx, ayy, azz, aidx

        return step

    for c in range(n // ch):
        if c == 0:
            axx = jnp.where(lane_ch == 0, x0, 0.0)
            ayy = jnp.where(lane_ch == 0, y0, 0.0)
            azz = jnp.where(lane_ch == 0, z0, 0.0)
            lo = 1
        else:
            axx = jnp.zeros((B, ch), jnp.float32)
            ayy = jnp.zeros((B, ch), jnp.float32)
            azz = jnp.zeros((B, ch), jnp.float32)
            lo = c * ch
        aidx = jnp.zeros((B, ch), jnp.int32)
        d, axx, ayy, azz, aidx = jax.lax.fori_loop(
            lo, (c + 1) * ch, step_fn(c), (d, axx, ayy, azz, aidx))
        sl = slice(c * ch, (c + 1) * ch)
        sx_ref[:, sl] = axx
        sy_ref[:, sl] = ayy
        sz_ref[:, sl] = azz
        if idx_ref is not None:
            idx_ref[:, sl] = aidx


def _fps_body(px, py, pz,
              idx2_ref, idx3_ref,
              l1x_ref, l1y_ref, l1z_ref,
              l2x_ref, l2y_ref, l2z_ref,
              l3x_ref, l3y_ref, l3z_ref):
    x, y, z = px[...], py[...], pz[...]
    _fps_level(x, y, z, N1, None, l1x_ref, l1y_ref, l1z_ref)
    x1, y1, z1 = l1x_ref[...], l1y_ref[...], l1z_ref[...]
    _fps_level(x1, y1, z1, N2, idx2_ref, l2x_ref, l2y_ref, l2z_ref)
    x2, y2, z2 = l2x_ref[...], l2y_ref[...], l2z_ref[...]
    _fps_level(x2, y2, z2, N3, idx3_ref, l3x_ref, l3y_ref, l3z_ref)


def _fps_call(px, py, pz):
    f32, i32 = jnp.float32, jnp.int32
    outs = [
        jax.ShapeDtypeStruct((B, N2), i32),
        jax.ShapeDtypeStruct((B, N3), i32),
        jax.ShapeDtypeStruct((B, N1), f32), jax.ShapeDtypeStruct((B, N1), f32),
        jax.ShapeDtypeStruct((B, N1), f32),
        jax.ShapeDtypeStruct((B, N2), f32), jax.ShapeDtypeStruct((B, N2), f32),
        jax.ShapeDtypeStruct((B, N2), f32),
        jax.ShapeDtypeStruct((B, N3), f32), jax.ShapeDtypeStruct((B, N3), f32),
        jax.ShapeDtypeStruct((B, N3), f32),
    ]
    return pl.pallas_call(_fps_body, out_shape=outs)(px, py, pz)


def _sa1_body(lx, ly, lz, w0, b0, w1, b1, w2, b2, out):
    h = lx[...] * w0[0:1, :] + ly[...] * w0[1:2, :] + lz[...] * w0[2:3, :] + b0[...]
    h = jnp.maximum(h, 0.0)
    h = jnp.maximum(_dot(h, w1[...]) + b1[...], 0.0)
    out[...] = jnp.maximum(_dot(h, w2[...]) + b2[...], 0.0)


def _sa1_call(lx, ly, lz, w0, b0, w1, b1, w2, b2):
    n = lx.shape[0]
    return pl.pallas_call(
        _sa1_body,
        out_shape=jax.ShapeDtypeStruct((n, 128), jnp.float32),
    )(lx, ly, lz, w0, b0, w1, b1, w2, b2)


def _sa_gather_body(idx, feat, qx, qy, qz,
                    w0f, w0p, b0, w1, b1, w2, b2, out, *, nsrc):
    nq = idx.shape[1]
    oneh = (_iota((1, nsrc), 1) == idx[0]).astype(jnp.float32)
    g = _dot(oneh, feat[0], _HI)
    posw = (qx[0] * w0p[0:1, :] + qy[0] * w0p[1:2, :] + qz[0] * w0p[2:3, :])
    h = jnp.maximum(_dot(g, w0f[...]) + posw + b0[...], 0.0)
    h = jnp.maximum(_dot(h, w1[...]) + b1[...], 0.0)
    out[0] = jnp.maximum(_dot(h, w2[...]) + b2[...], 0.0)


def _sa_gather_call(idx, feat, qx, qy, qz, w0f, w0p, b0, w1, b1, w2, b2):
    nq = idx.shape[1]
    nsrc, cin = feat.shape[1], feat.shape[2]
    cout = w2.shape[1]
    cmid1, cmid2 = w0f.shape[1], w1.shape[1]
    spec_w = lambda s: pl.BlockSpec(s, lambda b: (0,) * len(s))
    return pl.pallas_call(
        functools.partial(_sa_gather_body, nsrc=nsrc),
        grid=(B,),
        in_specs=[
            pl.BlockSpec((1, nq, 1), lambda b: (b, 0, 0)),
            pl.BlockSpec((1, nsrc, cin), lambda b: (b, 0, 0)),
            pl.BlockSpec((1, nq, 1), lambda b: (b, 0, 0)),
            pl.BlockSpec((1, nq, 1), lambda b: (b, 0, 0)),
            pl.BlockSpec((1, nq, 1), lambda b: (b, 0, 0)),
            spec_w((cin, cmid1)), spec_w((3, cmid1)), spec_w((1, cmid1)),
            spec_w((cmid1, cmid2)), spec_w((1, cmid2)),
            spec_w((cmid2, cout)), spec_w((1, cout)),
        ],
        out_specs=pl.BlockSpec((1, nq, cout), lambda b: (b, 0, 0)),
        out_shape=jax.ShapeDtypeStruct((B, nq, cout), jnp.float32),
    )(idx, feat, qx, qy, qz, w0f, w0p, b0, w1, b1, w2, b2)


def _knn_weights(qx, qy, qz, sx, sy, sz):
    dx, dy, dz = qx - sx, qy - sy, qz - sz
    d = dx * dx + dy * dy + dz * dz
    nq, ns = d.shape
    lane = _iota((1, ns), 1)
    mat = jnp.zeros((nq, ns), jnp.float32)
    wt = jnp.zeros((nq, 1), jnp.float32)
    for k in range(3):
        m = jnp.min(d, axis=1, keepdims=True)
        j = jnp.min(jnp.where(d == m, lane, ns), axis=1, keepdims=True)
        oneh = lane == j
        w = 1.0 / jnp.maximum(m, 1e-16)
        mat = jnp.where(oneh, w, mat)
        wt = wt + w
        if k < 2:
            d = jnp.where(oneh, jnp.inf, d)
    return mat, wt


def _fp_body(qx, qy, qz, sx, sy, sz, srcf, skipf,
             w0a, w0b, b0, w1, b1, out):
    mat, wt = _knn_weights(qx[0], qy[0], qz[0], sx[0], sy[0], sz[0])
    interp = _dot(mat, srcf[0]) / wt
    h = jnp.maximum(_dot(interp, w0a[...]) + _dot(skipf[0], w0b[...]) + b0[...], 0.0)
    out[0] = jnp.maximum(_dot(h, w1[...]) + b1[...], 0.0)


def _fp_call(qpos, spos, srcf, skipf, w0a, w0b, b0, w1, b1):
    qx, qy, qz = qpos
    sx, sy, sz = spos
    nq, ns = qx.shape[1], sx.shape[2]
    cin, cskip = srcf.shape[2], skipf.shape[2]
    cmid, cout = w0a.shape[1], w1.shape[1]
    spec_w = lambda s: pl.BlockSpec(s, lambda b: (0,) * len(s))
    return pl.pallas_call(
        _fp_body,
        grid=(B,),
        in_specs=[
            pl.BlockSpec((1, nq, 1), lambda b: (b, 0, 0)),
            pl.BlockSpec((1, nq, 1), lambda b: (b, 0, 0)),
            pl.BlockSpec((1, nq, 1), lambda b: (b, 0, 0)),
            pl.BlockSpec((1, 1, ns), lambda b: (b, 0, 0)),
            pl.BlockSpec((1, 1, ns), lambda b: (b, 0, 0)),
            pl.BlockSpec((1, 1, ns), lambda b: (b, 0, 0)),
            pl.BlockSpec((1, ns, cin), lambda b: (b, 0, 0)),
            pl.BlockSpec((1, nq, cskip), lambda b: (b, 0, 0)),
            spec_w((cin, cmid)), spec_w((cskip, cmid)), spec_w((1, cmid)),
            spec_w((cmid, cout)), spec_w((1, cout)),
        ],
        out_specs=pl.BlockSpec((1, nq, cout), lambda b: (b, 0, 0)),
        out_shape=jax.ShapeDtypeStruct((B, nq, cout), jnp.float32),
    )(qx, qy, qz, sx, sy, sz, srcf, skipf, w0a, w0b, b0, w1, b1)


def _fp1_body(qx, qy, qz, sx, sy, sz, srcf,
              w0f, w0p, b0, w1, b1, w2, b2, wh0, bh0, wh1, bh1, out):
    mat, wt = _knn_weights(qx[0], qy[0], qz[0], sx[0], sy[0], sz[0])
    interp = _dot(mat, srcf[0]) / wt
    posw = (qx[0] * w0p[0:1, :] + qy[0] * w0p[1:2, :] + qz[0] * w0p[2:3, :])
    h = jnp.maximum(_dot(interp, w0f[...]) + posw + b0[...], 0.0)
    h = jnp.maximum(_dot(h, w1[...]) + b1[...], 0.0)
    h = jnp.maximum(_dot(h, w2[...]) + b2[...], 0.0)
    a = jnp.maximum(_dot(h, wh0[...]) + bh0[...], 0.0)
    v = _dot(a, wh1[...]) + bh1[...]
    sp = jnp.maximum(v, 0.0) + jnp.log(1.0 + jnp.exp(-jnp.abs(v)))
    out[0] = sp + MIN_ALPHA


def _fp1_call(qpos, spos, srcf, wf, hd):
    qx, qy, qz = qpos
    sx, sy, sz = spos
    qt = 2048
    nblk = P // qt
    spec_w = lambda s: pl.BlockSpec(s, lambda b, j: (0,) * len(s))
    w0f, w0p, b0, w1, b1, w2, b2 = wf
    wh0, bh0, wh1, bh1 = hd
    return pl.pallas_call(
        _fp1_body,
        grid=(B, nblk),
        in_specs=[
            pl.BlockSpec((1, qt, 1), lambda b, j: (b, j, 0)),
            pl.BlockSpec((1, qt, 1), lambda b, j: (b, j, 0)),
            pl.BlockSpec((1, qt, 1), lambda b, j: (b, j, 0)),
            pl.BlockSpec((1, 1, N1), lambda b, j: (b, 0, 0)),
            pl.BlockSpec((1, 1, N1), lambda b, j: (b, 0, 0)),
            pl.BlockSpec((1, 1, N1), lambda b, j: (b, 0, 0)),
            pl.BlockSpec((1, N1, 128), lambda b, j: (b, 0, 0)),
            spec_w((128, 128)), spec_w((3, 128)), spec_w((1, 128)),
            spec_w((128, 128)), spec_w((1, 128)),
            spec_w((128, 128)), spec_w((1, 128)),
            spec_w((128, 64)), spec_w((1, 64)),
            spec_w((64, 1)), spec_w((1, 1)),
        ],
        out_specs=pl.BlockSpec((1, qt, 1), lambda b, j: (b, j, 0)),
        out_shape=jax.ShapeDtypeStruct((B, P, 1), jnp.float32),
    )(qx, qy, qz, sx, sy, sz, srcf,
      w0f, w0p, b0, w1, b1, w2, b2, wh0, bh0, wh1, bh1)


def kernel(pos, batch, params):
    del batch
    f32 = jnp.float32
    pos_b = pos.reshape(B, P, 3)
    px = pos_b[..., 0]
    py = pos_b[..., 1]
    pz = pos_b[..., 2]

    (idx2, idx3, l1x, l1y, l1z, l2x, l2y, l2z, l3x, l3y, l3z) = _fps_call(px, py, pz)

    rb = lambda b_: b_.reshape(1, -1).astype(f32)
    sa1 = params["sa1"]
    l1feat = _sa1_call(
        l1x.reshape(-1, 1), l1y.reshape(-1, 1), l1z.reshape(-1, 1),
        sa1["W"][0], rb(sa1["b"][0]), sa1["W"][1], rb(sa1["b"][1]),
        sa1["W"][2], rb(sa1["b"][2]),
    ).reshape(B, N1, 128)

    sa2 = params["sa2"]
    l2feat = _sa_gather_call(
        idx2.reshape(B, N2, 1), l1feat,
        l2x.reshape(B, N2, 1), l2y.reshape(B, N2, 1), l2z.reshape(B, N2, 1),
        sa2["W"][0][:128], sa2["W"][0][128:], rb(sa2["b"][0]),
        sa2["W"][1], rb(sa2["b"][1]), sa2["W"][2], rb(sa2["b"][2]),
    )

    sa3 = params["sa3"]
    l3feat = _sa_gather_call(
        idx3.reshape(B, N3, 1), l2feat,
        l3x.reshape(B, N3, 1), l3y.reshape(B, N3, 1), l3z.reshape(B, N3, 1),
        sa3["W"][0][:256], sa3["W"][0][256:], rb(sa3["b"][0]),
        sa3["W"][1], rb(sa3["b"][1]), sa3["W"][2], rb(sa3["b"][2]),
    )

    fp3 = params["fp3"]
    l2fp = _fp_call(
        (l2x.reshape(B, N2, 1), l2y.reshape(B, N2, 1), l2z.reshape(B, N2, 1)),
        (l3x.reshape(B, 1, N3), l3y.reshape(B, 1, N3), l3z.reshape(B, 1, N3)),
        l3feat, l2feat,
        fp3["W"][0][:1024], fp3["W"][0][1024:], rb(fp3["b"][0]),
        fp3["W"][1], rb(fp3["b"][1]),
    )

    fp2 = params["fp2"]
    l1fp = _fp_call(
        (l1x.reshape(B, N1, 1), l1y.reshape(B, N1, 1), l1z.reshape(B, N1, 1)),
        (l2x.reshape(B, 1, N2), l2y.reshape(B, 1, N2), l2z.reshape(B, 1, N2)),
        l2fp, l1feat,
        fp2["W"][0][:256], fp2["W"][0][256:], rb(fp2["b"][0]),
        fp2["W"][1], rb(fp2["b"][1]),
    )

    fp1 = params["fp1"]
    hd = params["head"]
    fp1_w = (fp1["W"][0][:128], fp1["W"][0][128:], rb(fp1["b"][0]),
             fp1["W"][1], rb(fp1["b"][1]), fp1["W"][2], rb(fp1["b"][2]))
    hd_w = (hd["W"][0], rb(hd["b"][0]), hd["W"][1], rb(hd["b"][1]))
    qpos0 = (pos_b[..., 0:1], pos_b[..., 1:2], pos_b[..., 2:3])
    spos1 = (l1x.reshape(B, 1, N1), l1y.reshape(B, 1, N1), l1z.reshape(B, 1, N1))
    alpha = _fp1_call(qpos0, spos1, l1fp, fp1_w, hd_w)

    alpha_mean_act = alpha.reshape(B, 1, P)
    alpha_std = jnp.full_like(alpha_mean_act, 0.01)
    return alpha_mean_act, alpha_std
